# baseline (speedup 1.0000x reference)
import functools

import jax
import jax.numpy as jnp
from jax.experimental import pallas as pl
from jax.experimental.pallas import tpu as pltpu

_NUM_CLASS = 8
_K = 20
_OBS_D = 16
_INIT_D = 24
_D2 = 16
_EMBED = 128


def _self_body(lab_ref, obs_ref, init_ref, wobs_ref, winit_ref, b_ref,
               out_ref):
    R = obs_ref.shape[0]
    lab = lab_ref[0, 0, :]

    obs_rep = pltpu.repeat(obs_ref[...], _NUM_CLASS, axis=1)
    slot = jax.lax.broadcasted_iota(jnp.int32, (R, _NUM_CLASS * _OBS_D), 1)
    obs_exp = jnp.where(slot // _OBS_D == lab[:, None], obs_rep, 0.0)
    obsfeat = jnp.dot(obs_exp, wobs_ref[...],
                      preferred_element_type=jnp.float32)

    rows = []
    for i in range(_NUM_CLASS):
        ci = jnp.dot(init_ref[i], winit_ref[i],
                     preferred_element_type=jnp.float32)
        rows.append(ci)
        rows.append(b_ref[i][None, :])
    ctab = jnp.concatenate(rows, axis=0)

    nrow = _NUM_CLASS * (_K + 1)
    c_iota = jax.lax.broadcasted_iota(jnp.int32, (R, _K, nrow), 2)
    k_iota = jax.lax.broadcasted_iota(jnp.int32, (R, _K, nrow), 1)
    tgt = lab[:, None, None] * (_K + 1)
    oh = ((c_iota == tgt + k_iota) | (c_iota == tgt + _K))
    initfeat = jnp.dot(oh.astype(jnp.float32).reshape(R * _K, nrow),
                       ctab, preferred_element_type=jnp.float32)
    out_ref[...] = obsfeat[:, None, :] + initfeat.reshape(R, _K, _EMBED)


def _nei_body(lab_ref, neis_ref, w_ref, b_ref, out_ref):
    R = neis_ref.shape[0]
    nc = _NUM_CLASS + 1
    f = neis_ref[...]
    eps = jnp.where(f >= 0, 1e-4, -1e-4).astype(jnp.float32)
    tn = 1.0 / (f + eps)
    lab = lab_ref[0, 0, :]
    tn_rep = pltpu.repeat(tn, nc, axis=1)
    slot = jax.lax.broadcasted_iota(jnp.int32, (R, nc * _D2), 1)
    e = jnp.where(slot // _D2 == lab[:, None], tn_rep, 0.0)
    acc = jnp.dot(e, w_ref[...], preferred_element_type=jnp.float32)
    cls = jax.lax.broadcasted_iota(jnp.int32, (R, nc), 1)
    onehot = (lab[:, None] == cls).astype(jnp.float32)
    acc = acc + jnp.dot(onehot, b_ref[...],
                        preferred_element_type=jnp.float32)
    out_ref[...] = acc


@functools.partial(jax.jit, static_argnames=())
def kernel(obs, neis, self_labels, nei_labels, init_trajs, W_self, b_self,
           W_nei, b_nei):
    B = obs.shape[0]
    N = neis.shape[1]
    nc = _NUM_CLASS + 1

    obs2 = obs.reshape(B, _OBS_D)
    init3 = init_trajs.reshape(_NUM_CLASS, _K, _INIT_D)
    wobs = jnp.transpose(W_self[:, :, :_OBS_D], (0, 2, 1)).reshape(
        _NUM_CLASS * _OBS_D, _EMBED)
    winit = jnp.transpose(W_self[:, :, _OBS_D:], (0, 2, 1))
    neis2 = neis.reshape(B * N, _D2)
    wnei = jnp.transpose(W_nei, (0, 2, 1)).reshape(nc * _D2, _EMBED)

    R1 = 256
    nb1 = B // R1
    lab1 = self_labels.reshape(nb1, 1, R1)
    x = pl.pallas_call(
        _self_body,
        grid=(nb1,),
        in_specs=[
            pl.BlockSpec((1, 1, R1), lambda i: (i, 0, 0)),
            pl.BlockSpec((R1, _OBS_D), lambda i: (i, 0)),
            pl.BlockSpec((_NUM_CLASS, _K, _INIT_D), lambda i: (0, 0, 0)),
            pl.BlockSpec((_NUM_CLASS * _OBS_D, _EMBED), lambda i: (0, 0)),
            pl.BlockSpec((_NUM_CLASS, _INIT_D, _EMBED), lambda i: (0, 0, 0)),
            pl.BlockSpec((_NUM_CLASS, _EMBED), lambda i: (0, 0)),
        ],
        out_specs=pl.BlockSpec((R1, _K, _EMBED), lambda i: (i, 0, 0)),
        out_shape=jax.ShapeDtypeStruct((B, _K, _EMBED), jnp.float32),
        compiler_params=pltpu.CompilerParams(
            dimension_semantics=("parallel",)),
    )(lab1, obs2, init3, wobs, winit, b_self)

    R2 = 2048
    nb2 = (B * N) // R2
    lab2 = nei_labels.reshape(nb2, 1, R2)
    nei_feats = pl.pallas_call(
        _nei_body,
        grid=(nb2,),
        in_specs=[
            pl.BlockSpec((1, 1, R2), lambda i: (i, 0, 0)),
            pl.BlockSpec((R2, _D2), lambda i: (i, 0)),
            pl.BlockSpec((nc * _D2, _EMBED), lambda i: (0, 0)),
            pl.BlockSpec((nc, _EMBED), lambda i: (0, 0)),
        ],
        out_specs=pl.BlockSpec((R2, _EMBED), lambda i: (i, 0)),
        out_shape=jax.ShapeDtypeStruct((B * N, _EMBED), jnp.float32),
        compiler_params=pltpu.CompilerParams(
            dimension_semantics=("parallel",)),
    )(lab2, neis2, wnei, b_nei)

    return (x, nei_feats.reshape(B, N, _EMBED))


# speedup vs baseline: 1.2553x; 1.2553x over previous
import functools

import jax
import jax.numpy as jnp
from jax.experimental import pallas as pl
from jax.experimental.pallas import tpu as pltpu

_NUM_CLASS = 8
_K = 20
_OBS_D = 16
_INIT_D = 24
_D2 = 16
_EMBED = 128


def _self_body(lab_ref, obs_ref, init_ref, wobs_ref, winit_ref, b_ref,
               out_ref):
    R = obs_ref.shape[0]
    lab = lab_ref[0, 0, :]

    obs_rep = pltpu.repeat(obs_ref[...], _NUM_CLASS, axis=1)
    slot = jax.lax.broadcasted_iota(jnp.int32, (R, _NUM_CLASS * _OBS_D), 1)
    obs_exp = jnp.where(slot // _OBS_D == lab[:, None], obs_rep, 0.0)
    obsfeat = jnp.dot(obs_exp, wobs_ref[...],
                      preferred_element_type=jnp.float32)

    rows = []
    for i in range(_NUM_CLASS):
        ci = jnp.dot(init_ref[i], winit_ref[i],
                     preferred_element_type=jnp.float32)
        rows.append(ci)
        rows.append(b_ref[i][None, :])
    ctab = jnp.concatenate(rows, axis=0)

    nrow = _NUM_CLASS * (_K + 1)
    c_iota = jax.lax.broadcasted_iota(jnp.int32, (R, _K, nrow), 2)
    k_iota = jax.lax.broadcasted_iota(jnp.int32, (R, _K, nrow), 1)
    tgt = lab[:, None, None] * (_K + 1)
    oh = ((c_iota == tgt + k_iota) | (c_iota == tgt + _K))
    initfeat = jnp.dot(oh.astype(jnp.float32).reshape(R * _K, nrow),
                       ctab, preferred_element_type=jnp.float32)
    out_ref[...] = obsfeat[:, None, :] + initfeat.reshape(R, _K, _EMBED)


def _nei_body(lab_ref, neis_ref, w_ref, b_ref, out_ref):
    R = neis_ref.shape[0]
    nc = _NUM_CLASS + 1
    f = neis_ref[...]
    eps = jnp.where(f >= 0, 1e-4, -1e-4).astype(jnp.float32)
    tn = 1.0 / (f + eps)
    lab = lab_ref[0, 0, :]
    labc = lab[:, None]
    cls = jax.lax.broadcasted_iota(jnp.int32, (R, nc), 1)
    onehot = (labc == cls).astype(jnp.float32)
    acc = jnp.dot(onehot, b_ref[...], preferred_element_type=jnp.float32)
    w3 = w_ref[...].reshape(nc, _D2, _EMBED)
    for i in range(nc):
        ti = jnp.where(labc == i, tn, 0.0)
        acc = acc + jnp.dot(ti, w3[i], preferred_element_type=jnp.float32)
    out_ref[...] = acc


@functools.partial(jax.jit, static_argnames=())
def kernel(obs, neis, self_labels, nei_labels, init_trajs, W_self, b_self,
           W_nei, b_nei):
    B = obs.shape[0]
    N = neis.shape[1]
    nc = _NUM_CLASS + 1

    obs2 = obs.reshape(B, _OBS_D)
    init3 = init_trajs.reshape(_NUM_CLASS, _K, _INIT_D)
    wobs = jnp.transpose(W_self[:, :, :_OBS_D], (0, 2, 1)).reshape(
        _NUM_CLASS * _OBS_D, _EMBED)
    winit = jnp.transpose(W_self[:, :, _OBS_D:], (0, 2, 1))
    neis2 = neis.reshape(B * N, _D2)
    wnei = jnp.transpose(W_nei, (0, 2, 1)).reshape(nc * _D2, _EMBED)

    R1 = 256
    nb1 = B // R1
    lab1 = self_labels.reshape(nb1, 1, R1)
    x = pl.pallas_call(
        _self_body,
        grid=(nb1,),
        in_specs=[
            pl.BlockSpec((1, 1, R1), lambda i: (i, 0, 0)),
            pl.BlockSpec((R1, _OBS_D), lambda i: (i, 0)),
            pl.BlockSpec((_NUM_CLASS, _K, _INIT_D), lambda i: (0, 0, 0)),
            pl.BlockSpec((_NUM_CLASS * _OBS_D, _EMBED), lambda i: (0, 0)),
            pl.BlockSpec((_NUM_CLASS, _INIT_D, _EMBED), lambda i: (0, 0, 0)),
            pl.BlockSpec((_NUM_CLASS, _EMBED), lambda i: (0, 0)),
        ],
        out_specs=pl.BlockSpec((R1, _K, _EMBED), lambda i: (i, 0, 0)),
        out_shape=jax.ShapeDtypeStruct((B, _K, _EMBED), jnp.float32),
        compiler_params=pltpu.CompilerParams(
            dimension_semantics=("parallel",)),
    )(lab1, obs2, init3, wobs, winit, b_self)

    R2 = 2048
    nb2 = (B * N) // R2
    lab2 = nei_labels.reshape(nb2, 1, R2)
    nei_feats = pl.pallas_call(
        _nei_body,
        grid=(nb2,),
        in_specs=[
            pl.BlockSpec((1, 1, R2), lambda i: (i, 0, 0)),
            pl.BlockSpec((R2, _D2), lambda i: (i, 0)),
            pl.BlockSpec((nc * _D2, _EMBED), lambda i: (0, 0)),
            pl.BlockSpec((nc, _EMBED), lambda i: (0, 0)),
        ],
        out_specs=pl.BlockSpec((R2, _EMBED), lambda i: (i, 0)),
        out_shape=jax.ShapeDtypeStruct((B * N, _EMBED), jnp.float32),
        compiler_params=pltpu.CompilerParams(
            dimension_semantics=("parallel",)),
    )(lab2, neis2, wnei, b_nei)

    return (x, nei_feats.reshape(B, N, _EMBED))


# speedup vs baseline: 1.2634x; 1.0065x over previous
import functools

import jax
import jax.numpy as jnp
from jax.experimental import pallas as pl
from jax.experimental.pallas import tpu as pltpu

_NUM_CLASS = 8
_K = 20
_OBS_D = 16
_INIT_D = 24
_D2 = 16
_EMBED = 128
_R1 = 256
_R2 = 4096
_NB1 = 4096 // _R1


def _fused_body(lab2_ref, neis_ref, wnei_ref, bnei_ref,
                lab1_ref, obs_ref, init_ref, wobs_ref, winit_ref, bself_ref,
                nei_out_ref, x_out_ref):
    step = pl.program_id(0)
    nc = _NUM_CLASS + 1

    R = neis_ref.shape[0]
    f = neis_ref[...]
    eps = jnp.where(f >= 0, 1e-4, -1e-4).astype(jnp.float32)
    tn = 1.0 / (f + eps)
    lab = lab2_ref[0, 0, :]
    labc = lab[:, None]
    cls = jax.lax.broadcasted_iota(jnp.int32, (R, nc), 1)
    onehot = (labc == cls).astype(jnp.float32)
    acc = jnp.dot(onehot, bnei_ref[...], preferred_element_type=jnp.float32)
    w3 = wnei_ref[...].reshape(nc, _D2, _EMBED)
    for i in range(nc):
        ti = jnp.where(labc == i, tn, 0.0)
        acc = acc + jnp.dot(ti, w3[i], preferred_element_type=jnp.float32)
    nei_out_ref[...] = acc

    @pl.when(step < _NB1)
    def _self_stage():
        Rs = obs_ref.shape[0]
        slab = lab1_ref[0, 0, :]

        obs_rep = pltpu.repeat(obs_ref[...], _NUM_CLASS, axis=1)
        slot = jax.lax.broadcasted_iota(
            jnp.int32, (Rs, _NUM_CLASS * _OBS_D), 1)
        obs_exp = jnp.where(slot // _OBS_D == slab[:, None], obs_rep, 0.0)
        obsfeat = jnp.dot(obs_exp, wobs_ref[...],
                          preferred_element_type=jnp.float32)

        rows = []
        for i in range(_NUM_CLASS):
            ci = jnp.dot(init_ref[i], winit_ref[i],
                         preferred_element_type=jnp.float32)
            rows.append(ci)
            rows.append(bself_ref[i][None, :])
        ctab = jnp.concatenate(rows, axis=0)

        nrow = _NUM_CLASS * (_K + 1)
        c_iota = jax.lax.broadcasted_iota(jnp.int32, (Rs, _K, nrow), 2)
        k_iota = jax.lax.broadcasted_iota(jnp.int32, (Rs, _K, nrow), 1)
        tgt = slab[:, None, None] * (_K + 1)
        oh = ((c_iota == tgt + k_iota) | (c_iota == tgt + _K))
        initfeat = jnp.dot(oh.astype(jnp.float32).reshape(Rs * _K, nrow),
                           ctab, preferred_element_type=jnp.float32)
        x_out_ref[...] = obsfeat[:, None, :] + initfeat.reshape(
            Rs, _K, _EMBED)


@functools.partial(jax.jit, static_argnames=())
def kernel(obs, neis, self_labels, nei_labels, init_trajs, W_self, b_self,
           W_nei, b_nei):
    B = obs.shape[0]
    N = neis.shape[1]
    nc = _NUM_CLASS + 1

    obs2 = obs.reshape(B, _OBS_D)
    init3 = init_trajs.reshape(_NUM_CLASS, _K, _INIT_D)
    wobs = jnp.transpose(W_self[:, :, :_OBS_D], (0, 2, 1)).reshape(
        _NUM_CLASS * _OBS_D, _EMBED)
    winit = jnp.transpose(W_self[:, :, _OBS_D:], (0, 2, 1))
    neis2 = neis.reshape(B * N, _D2)
    wnei = jnp.transpose(W_nei, (0, 2, 1)).reshape(nc * _D2, _EMBED)

    nb1 = B // _R1
    nb2 = (B * N) // _R2
    lab1 = self_labels.reshape(nb1, 1, _R1)
    lab2 = nei_labels.reshape(nb2, 1, _R2)

    def clamp1(i):
        return (jnp.minimum(i, nb1 - 1), 0)

    def clamp1_3(i):
        return (jnp.minimum(i, nb1 - 1), 0, 0)

    nei_feats, x = pl.pallas_call(
        _fused_body,
        grid=(nb2,),
        in_specs=[
            pl.BlockSpec((1, 1, _R2), lambda i: (i, 0, 0)),
            pl.BlockSpec((_R2, _D2), lambda i: (i, 0)),
            pl.BlockSpec((nc * _D2, _EMBED), lambda i: (0, 0)),
            pl.BlockSpec((nc, _EMBED), lambda i: (0, 0)),
            pl.BlockSpec((1, 1, _R1), clamp1_3),
            pl.BlockSpec((_R1, _OBS_D), clamp1),
            pl.BlockSpec((_NUM_CLASS, _K, _INIT_D), lambda i: (0, 0, 0)),
            pl.BlockSpec((_NUM_CLASS * _OBS_D, _EMBED), lambda i: (0, 0)),
            pl.BlockSpec((_NUM_CLASS, _INIT_D, _EMBED), lambda i: (0, 0, 0)),
            pl.BlockSpec((_NUM_CLASS, _EMBED), lambda i: (0, 0)),
        ],
        out_specs=[
            pl.BlockSpec((_R2, _EMBED), lambda i: (i, 0)),
            pl.BlockSpec((_R1, _K, _EMBED), clamp1_3),
        ],
        out_shape=[
            jax.ShapeDtypeStruct((B * N, _EMBED), jnp.float32),
            jax.ShapeDtypeStruct((B, _K, _EMBED), jnp.float32),
        ],
    )(lab2, neis2, wnei, b_nei, lab1, obs2, init3, wobs, winit, b_self)

    return (x, nei_feats.reshape(B, N, _EMBED))


# speedup vs baseline: 1.2774x; 1.0111x over previous
import functools

import jax
import jax.numpy as jnp
from jax.experimental import pallas as pl
from jax.experimental.pallas import tpu as pltpu

_NUM_CLASS = 8
_K = 20
_OBS_D = 16
_INIT_D = 24
_D2 = 16
_EMBED = 128


def _self_body(lab_ref, obs_ref, init_ref, wobs_ref, winit_ref, b_ref,
               out_ref):
    R = obs_ref.shape[0]
    lab = lab_ref[0, 0, :]

    obs_rep = pltpu.repeat(obs_ref[...], _NUM_CLASS, axis=1)
    slot = jax.lax.broadcasted_iota(jnp.int32, (R, _NUM_CLASS * _OBS_D), 1)
    obs_exp = jnp.where(slot // _OBS_D == lab[:, None], obs_rep, 0.0)
    obsfeat = jnp.dot(obs_exp, wobs_ref[...],
                      preferred_element_type=jnp.float32)

    rows = []
    for i in range(_NUM_CLASS):
        ci = jnp.dot(init_ref[i], winit_ref[i],
                     preferred_element_type=jnp.float32)
        rows.append(ci)
        rows.append(b_ref[i][None, :])
    ctab = jnp.concatenate(rows, axis=0)

    nrow = _NUM_CLASS * (_K + 1)
    c_iota = jax.lax.broadcasted_iota(jnp.int32, (R, _K, nrow), 2)
    k_iota = jax.lax.broadcasted_iota(jnp.int32, (R, _K, nrow), 1)
    tgt = lab[:, None, None] * (_K + 1)
    oh = ((c_iota == tgt + k_iota) | (c_iota == tgt + _K))
    initfeat = jnp.dot(oh.astype(jnp.float32).reshape(R * _K, nrow),
                       ctab, preferred_element_type=jnp.float32)
    out_ref[...] = obsfeat[:, None, :] + initfeat.reshape(R, _K, _EMBED)


def _nei_body(lab_ref, neis_ref, w_ref, b_ref, out_ref):
    R = neis_ref.shape[0]
    nc = _NUM_CLASS + 1
    f = neis_ref[...]
    eps = jnp.where(f >= 0, 1e-4, -1e-4).astype(jnp.float32)
    tn = 1.0 / (f + eps)
    lab = lab_ref[0, 0, :]
    labc = lab[:, None]
    cls = jax.lax.broadcasted_iota(jnp.int32, (R, nc), 1)
    onehot = (labc == cls).astype(jnp.float32)
    acc = jnp.dot(onehot, b_ref[...], preferred_element_type=jnp.float32)
    w3 = w_ref[...].reshape(nc, _D2, _EMBED)
    for i in range(nc):
        ti = jnp.where(labc == i, tn, 0.0)
        acc = acc + jnp.dot(ti, w3[i], preferred_element_type=jnp.float32)
    out_ref[...] = acc


@functools.partial(jax.jit, static_argnames=())
def kernel(obs, neis, self_labels, nei_labels, init_trajs, W_self, b_self,
           W_nei, b_nei):
    B = obs.shape[0]
    N = neis.shape[1]
    nc = _NUM_CLASS + 1

    obs2 = obs.reshape(B, _OBS_D)
    init3 = init_trajs.reshape(_NUM_CLASS, _K, _INIT_D)
    wobs = jnp.transpose(W_self[:, :, :_OBS_D], (0, 2, 1)).reshape(
        _NUM_CLASS * _OBS_D, _EMBED)
    winit = jnp.transpose(W_self[:, :, _OBS_D:], (0, 2, 1))
    neis2 = neis.reshape(B * N, _D2)
    wnei = jnp.transpose(W_nei, (0, 2, 1)).reshape(nc * _D2, _EMBED)

    R1 = 256
    nb1 = B // R1
    lab1 = self_labels.reshape(nb1, 1, R1)
    x = pl.pallas_call(
        _self_body,
        grid=(nb1,),
        in_specs=[
            pl.BlockSpec((1, 1, R1), lambda i: (i, 0, 0)),
            pl.BlockSpec((R1, _OBS_D), lambda i: (i, 0)),
            pl.BlockSpec((_NUM_CLASS, _K, _INIT_D), lambda i: (0, 0, 0)),
            pl.BlockSpec((_NUM_CLASS * _OBS_D, _EMBED), lambda i: (0, 0)),
            pl.BlockSpec((_NUM_CLASS, _INIT_D, _EMBED), lambda i: (0, 0, 0)),
            pl.BlockSpec((_NUM_CLASS, _EMBED), lambda i: (0, 0)),
        ],
        out_specs=pl.BlockSpec((R1, _K, _EMBED), lambda i: (i, 0, 0)),
        out_shape=jax.ShapeDtypeStruct((B, _K, _EMBED), jnp.float32),
        compiler_params=pltpu.CompilerParams(
            dimension_semantics=("parallel",)),
    )(lab1, obs2, init3, wobs, winit, b_self)

    R2 = 4096
    nb2 = (B * N) // R2
    lab2 = nei_labels.reshape(nb2, 1, R2)
    nei_feats = pl.pallas_call(
        _nei_body,
        grid=(nb2,),
        in_specs=[
            pl.BlockSpec((1, 1, R2), lambda i: (i, 0, 0)),
            pl.BlockSpec((R2, _D2), lambda i: (i, 0)),
            pl.BlockSpec((nc * _D2, _EMBED), lambda i: (0, 0)),
            pl.BlockSpec((nc, _EMBED), lambda i: (0, 0)),
        ],
        out_specs=pl.BlockSpec((R2, _EMBED), lambda i: (i, 0)),
        out_shape=jax.ShapeDtypeStruct((B * N, _EMBED), jnp.float32),
        compiler_params=pltpu.CompilerParams(
            dimension_semantics=("parallel",)),
    )(lab2, neis2, wnei, b_nei)

    return (x, nei_feats.reshape(B, N, _EMBED))


# speedup vs baseline: 1.3439x; 1.0520x over previous
import functools

import jax
import jax.numpy as jnp
from jax.experimental import pallas as pl
from jax.experimental.pallas import tpu as pltpu

_NUM_CLASS = 8
_K = 20
_OBS_D = 16
_INIT_D = 24
_D2 = 16
_EMBED = 128


def _self_body(lab_ref, obs_ref, init_ref, wobs_ref, winit_ref, b_ref,
               out_ref):
    R = obs_ref.shape[0]
    lab = lab_ref[0, 0, :]

    obs_rep = pltpu.repeat(obs_ref[...], _NUM_CLASS, axis=1)
    slot = jax.lax.broadcasted_iota(jnp.int32, (R, _NUM_CLASS * _OBS_D), 1)
    obs_exp = jnp.where(slot // _OBS_D == lab[:, None], obs_rep, 0.0)
    obsfeat = jnp.dot(obs_exp, wobs_ref[...],
                      preferred_element_type=jnp.float32)

    rows = []
    for i in range(_NUM_CLASS):
        ci = jnp.dot(init_ref[i], winit_ref[i],
                     preferred_element_type=jnp.float32)
        rows.append(ci)
        rows.append(b_ref[i][None, :])
    ctab = jnp.concatenate(rows, axis=0)

    nrow = _NUM_CLASS * (_K + 1)
    c_iota = jax.lax.broadcasted_iota(jnp.int32, (R, _K, nrow), 2)
    k_iota = jax.lax.broadcasted_iota(jnp.int32, (R, _K, nrow), 1)
    tgt = lab[:, None, None] * (_K + 1)
    oh = ((c_iota == tgt + k_iota) | (c_iota == tgt + _K))
    initfeat = jnp.dot(oh.astype(jnp.float32).reshape(R * _K, nrow),
                       ctab, preferred_element_type=jnp.float32)
    out_ref[...] = obsfeat[:, None, :] + initfeat.reshape(R, _K, _EMBED)


def _nei_body(lab_ref, neis_ref, w_ref, b_ref, out_ref):
    R = neis_ref.shape[0] * neis_ref.shape[1]
    nc = _NUM_CLASS + 1
    f = neis_ref[...].reshape(R, _D2)
    eps = jnp.where(f >= 0, 1e-4, -1e-4).astype(jnp.float32)
    tn = 1.0 / (f + eps)
    lab = lab_ref[0, 0, :]
    labc = lab[:, None]
    cls = jax.lax.broadcasted_iota(jnp.int32, (R, nc), 1)
    onehot = (labc == cls).astype(jnp.float32)
    acc = jnp.dot(onehot, b_ref[...], preferred_element_type=jnp.float32)
    w3 = w_ref[...].reshape(nc, _D2, _EMBED)
    for i in range(nc):
        ti = jnp.where(labc == i, tn, 0.0)
        acc = acc + jnp.dot(ti, w3[i], preferred_element_type=jnp.float32)
    out_ref[...] = acc


@functools.partial(jax.jit, static_argnames=())
def kernel(obs, neis, self_labels, nei_labels, init_trajs, W_self, b_self,
           W_nei, b_nei):
    B = obs.shape[0]
    N = neis.shape[1]
    nc = _NUM_CLASS + 1

    obs2 = obs.reshape(B, _OBS_D)
    init3 = init_trajs.reshape(_NUM_CLASS, _K, _INIT_D)
    wobs = jnp.transpose(W_self[:, :, :_OBS_D], (0, 2, 1)).reshape(
        _NUM_CLASS * _OBS_D, _EMBED)
    winit = jnp.transpose(W_self[:, :, _OBS_D:], (0, 2, 1))
    neis3 = neis.reshape(B, N, _D2)
    wnei = jnp.transpose(W_nei, (0, 2, 1)).reshape(nc * _D2, _EMBED)

    R1 = 256
    nb1 = B // R1
    lab1 = self_labels.reshape(nb1, 1, R1)
    x = pl.pallas_call(
        _self_body,
        grid=(nb1,),
        in_specs=[
            pl.BlockSpec((1, 1, R1), lambda i: (i, 0, 0)),
            pl.BlockSpec((R1, _OBS_D), lambda i: (i, 0)),
            pl.BlockSpec((_NUM_CLASS, _K, _INIT_D), lambda i: (0, 0, 0)),
            pl.BlockSpec((_NUM_CLASS * _OBS_D, _EMBED), lambda i: (0, 0)),
            pl.BlockSpec((_NUM_CLASS, _INIT_D, _EMBED), lambda i: (0, 0, 0)),
            pl.BlockSpec((_NUM_CLASS, _EMBED), lambda i: (0, 0)),
        ],
        out_specs=pl.BlockSpec((R1, _K, _EMBED), lambda i: (i, 0, 0)),
        out_shape=jax.ShapeDtypeStruct((B, _K, _EMBED), jnp.float32),
        compiler_params=pltpu.CompilerParams(
            dimension_semantics=("parallel",)),
    )(lab1, obs2, init3, wobs, winit, b_self)

    R2 = 4096
    Rb = R2 // N
    nb2 = (B * N) // R2
    lab2 = nei_labels.reshape(nb2, 1, R2)
    nei_feats = pl.pallas_call(
        _nei_body,
        grid=(nb2,),
        in_specs=[
            pl.BlockSpec((1, 1, R2), lambda i: (i, 0, 0)),
            pl.BlockSpec((Rb, N, _D2), lambda i: (i, 0, 0)),
            pl.BlockSpec((nc * _D2, _EMBED), lambda i: (0, 0)),
            pl.BlockSpec((nc, _EMBED), lambda i: (0, 0)),
        ],
        out_specs=pl.BlockSpec((R2, _EMBED), lambda i: (i, 0)),
        out_shape=jax.ShapeDtypeStruct((B * N, _EMBED), jnp.float32),
        compiler_params=pltpu.CompilerParams(
            dimension_semantics=("parallel",)),
    )(lab2, neis3, wnei, b_nei)

    return (x, nei_feats.reshape(B, N, _EMBED))
